# Initial kernel scaffold; baseline (speedup 1.0000x reference)
import jax
import jax.numpy as jnp
from jax.experimental import pallas as pl


def kernel(v, e, edge_index, W1, b1, W2, b2):
    raise NotImplementedError("write your pallas kernel here")


# speedup vs baseline: 4.0673x; 4.0673x over previous
import functools

import jax
import jax.numpy as jnp
import numpy as np
from jax import lax
from jax.experimental import pallas as pl
from jax.experimental.pallas import tpu as pltpu
from jax.experimental.pallas import tpu_sc as plsc

N = 10000
E = 320000
F = 128
H = 128

NC = 2
NS = 16
NW = NC * NS
EPT = E // NW
C = 40
PAIRS = EPT // (2 * C)
RPT = 632
RPT_LAST = N - (NS - 1) * RPT

_LOG2 = float(np.log(2.0))


def _sc_segment_sum(e, idx, zrows):
    mesh = plsc.VectorSubcoreMesh(core_axis_name="c", subcore_axis_name="s")

    @functools.partial(
        pl.kernel,
        mesh=mesh,
        out_type=jax.ShapeDtypeStruct((NC, N, F), jnp.float32),
        scratch_types=[
            pltpu.VMEM_SHARED((N, F), jnp.float32),
            pltpu.VMEM((C, F), jnp.float32),
            pltpu.VMEM((C, F), jnp.float32),
            pltpu.VMEM((C,), jnp.int32),
            pltpu.VMEM((C,), jnp.int32),
            pltpu.SemaphoreType.DMA,
            pltpu.SemaphoreType.DMA,
        ],
    )
    def seg_sum(e_hbm, idx_hbm, z_hbm, out_hbm, acc, eb0, eb1, ib0, ib1, s0, s1):
        c = lax.axis_index("c")
        s = lax.axis_index("s")
        wid = s * NC + c
        ebase = wid * EPT

        off = pl.multiple_of(s * RPT, 8)

        @pl.when(s < NS - 1)
        def _():
            pltpu.sync_copy(z_hbm, acc.at[pl.ds(off, RPT)])

        @pl.when(s == NS - 1)
        def _():
            pltpu.sync_copy(
                z_hbm.at[pl.ds(0, RPT_LAST)], acc.at[pl.ds(off, RPT_LAST)]
            )

        pltpu.async_copy(e_hbm.at[pl.ds(ebase, C)], eb0, s0)
        pltpu.async_copy(idx_hbm.at[pl.ds(ebase, C)], ib0, s0)

        plsc.subcore_barrier()

        def body(g, carry):
            b0 = ebase + (2 * g) * C
            b1 = b0 + C
            pltpu.make_async_copy(e_hbm.at[pl.ds(b0, C)], eb0, s0).wait()
            pltpu.make_async_copy(idx_hbm.at[pl.ds(b0, C)], ib0, s0).wait()
            pltpu.async_copy(e_hbm.at[pl.ds(b1, C)], eb1, s1)
            pltpu.async_copy(idx_hbm.at[pl.ds(b1, C)], ib1, s1)
            pltpu.sync_copy(eb0, acc.at[ib0], add=True)
            pltpu.make_async_copy(e_hbm.at[pl.ds(b1, C)], eb1, s1).wait()
            pltpu.make_async_copy(idx_hbm.at[pl.ds(b1, C)], ib1, s1).wait()

            @pl.when(g + 1 < PAIRS)
            def _():
                b2 = b1 + C
                pltpu.async_copy(e_hbm.at[pl.ds(b2, C)], eb0, s0)
                pltpu.async_copy(idx_hbm.at[pl.ds(b2, C)], ib0, s0)

            pltpu.sync_copy(eb1, acc.at[ib1], add=True)
            return carry

        lax.fori_loop(0, PAIRS, body, 0)

        plsc.subcore_barrier()

        @pl.when(s < NS - 1)
        def _():
            pltpu.sync_copy(
                acc.at[pl.ds(off, RPT)], out_hbm.at[c, pl.ds(off, RPT)]
            )

        @pl.when(s == NS - 1)
        def _():
            pltpu.sync_copy(
                acc.at[pl.ds(off, RPT_LAST)],
                out_hbm.at[c, pl.ds(off, RPT_LAST)],
            )

    return seg_sum(e, idx, zrows)


def _mlp(p, v, w1t, b1r, w2t, b2r):
    bn = 1000

    def body(p_ref, v_ref, w1_ref, b1_ref, w2_ref, b2_ref, o_ref):
        ssum = p_ref[0] + p_ref[1]
        h = jnp.dot(ssum, w1_ref[...], preferred_element_type=jnp.float32)
        h = h + b1_ref[...]
        sp = jnp.maximum(h, 0.0) + jnp.log1p(jnp.exp(-jnp.abs(h))) - _LOG2
        o = jnp.dot(sp, w2_ref[...], preferred_element_type=jnp.float32)
        o_ref[...] = o + b2_ref[...] + v_ref[...]

    return pl.pallas_call(
        body,
        grid=(N // bn,),
        in_specs=[
            pl.BlockSpec((NC, bn, H), lambda i: (0, i, 0)),
            pl.BlockSpec((bn, H), lambda i: (i, 0)),
            pl.BlockSpec((H, H), lambda i: (0, 0)),
            pl.BlockSpec((1, H), lambda i: (0, 0)),
            pl.BlockSpec((H, H), lambda i: (0, 0)),
            pl.BlockSpec((1, H), lambda i: (0, 0)),
        ],
        out_specs=pl.BlockSpec((bn, H), lambda i: (i, 0)),
        out_shape=jax.ShapeDtypeStruct((N, H), jnp.float32),
    )(p, v, w1t, b1r, w2t, b2r)


def kernel(v, e, edge_index, W1, b1, W2, b2):
    idx = edge_index[1]
    zrows = jnp.zeros((RPT, F), jnp.float32)
    partials = _sc_segment_sum(e, idx, zrows)
    return _mlp(
        partials,
        v,
        W1.T,
        b1.reshape(1, H),
        W2.T,
        b2.reshape(1, H),
    )


# speedup vs baseline: 7.7786x; 1.9125x over previous
import functools

import jax
import jax.numpy as jnp
import numpy as np
from jax import lax
from jax.experimental import pallas as pl
from jax.experimental.pallas import tpu as pltpu
from jax.experimental.pallas import tpu_sc as plsc

N = 10000
E = 320000
F = 128
H = 128

NC = 2
NS = 16
NW = NC * NS
EPT = E // NW
C = 80
NCH = EPT // C
NBUF = 3
LOOKAHEAD = 2
NOUT = NCH // NBUF
NREM = NCH - NOUT * NBUF
RPT = 632
RPT_LAST = N - (NS - 1) * RPT

_LOG2 = float(np.log(2.0))


def _sc_segment_sum(e, idx, zrows):
    mesh = plsc.VectorSubcoreMesh(core_axis_name="c", subcore_axis_name="s")

    @functools.partial(
        pl.kernel,
        mesh=mesh,
        out_type=jax.ShapeDtypeStruct((NC, N, F), jnp.float32),
        scratch_types=[
            pltpu.VMEM_SHARED((N, F), jnp.float32),
            pltpu.VMEM((NCH, C), jnp.int32),
            [pltpu.VMEM((C, F), jnp.float32) for _ in range(NBUF)],
            [pltpu.SemaphoreType.DMA for _ in range(NBUF)],
        ],
    )
    def seg_sum(e_hbm, idx_hbm, z_hbm, out_hbm, acc, ibuf, ebufs, sems):
        c = lax.axis_index("c")
        s = lax.axis_index("s")
        wid = s * NC + c
        ebase = wid * EPT

        pltpu.sync_copy(idx_hbm.at[wid], ibuf)

        off = pl.multiple_of(s * RPT, 8)

        @pl.when(s < NS - 1)
        def _():
            pltpu.sync_copy(z_hbm, acc.at[pl.ds(off, RPT)])

        @pl.when(s == NS - 1)
        def _():
            pltpu.sync_copy(
                z_hbm.at[pl.ds(0, RPT_LAST)], acc.at[pl.ds(off, RPT_LAST)]
            )

        for b in range(LOOKAHEAD):
            pltpu.async_copy(
                e_hbm.at[pl.ds(ebase + b * C, C)], ebufs[b], sems[b]
            )

        plsc.subcore_barrier()

        def process_chunk(j, b, may_prefetch):
            pltpu.make_async_copy(
                e_hbm.at[pl.ds(ebase + j * C, C)], ebufs[b], sems[b]
            ).wait()
            if may_prefetch:
                @pl.when(j + LOOKAHEAD < NCH)
                def _():
                    jn = j + LOOKAHEAD
                    bn = (b + LOOKAHEAD) % NBUF
                    pltpu.async_copy(
                        e_hbm.at[pl.ds(ebase + jn * C, C)], ebufs[bn], sems[bn]
                    )
            pltpu.sync_copy(ebufs[b], acc.at[ibuf.at[j]], add=True)

        def body(g, carry):
            for b in range(NBUF):
                process_chunk(g * NBUF + b, b, True)
            return carry

        lax.fori_loop(0, NOUT, body, 0)
        for r in range(NREM):
            j = NOUT * NBUF + r
            process_chunk(j, j % NBUF, j + LOOKAHEAD < NCH)

        plsc.subcore_barrier()

        @pl.when(s < NS - 1)
        def _():
            pltpu.sync_copy(
                acc.at[pl.ds(off, RPT)], out_hbm.at[c, pl.ds(off, RPT)]
            )

        @pl.when(s == NS - 1)
        def _():
            pltpu.sync_copy(
                acc.at[pl.ds(off, RPT_LAST)],
                out_hbm.at[c, pl.ds(off, RPT_LAST)],
            )

    return seg_sum(e, idx, zrows)


def _mlp(p, v, w1t, b1r, w2t, b2r):
    bn = 1000

    def body(p_ref, v_ref, w1_ref, b1_ref, w2_ref, b2_ref, o_ref):
        ssum = p_ref[0] + p_ref[1]
        h = jnp.dot(ssum, w1_ref[...], preferred_element_type=jnp.float32)
        h = h + b1_ref[...]
        sp = jnp.maximum(h, 0.0) + jnp.log1p(jnp.exp(-jnp.abs(h))) - _LOG2
        o = jnp.dot(sp, w2_ref[...], preferred_element_type=jnp.float32)
        o_ref[...] = o + b2_ref[...] + v_ref[...]

    return pl.pallas_call(
        body,
        grid=(N // bn,),
        in_specs=[
            pl.BlockSpec((NC, bn, H), lambda i: (0, i, 0)),
            pl.BlockSpec((bn, H), lambda i: (i, 0)),
            pl.BlockSpec((H, H), lambda i: (0, 0)),
            pl.BlockSpec((1, H), lambda i: (0, 0)),
            pl.BlockSpec((H, H), lambda i: (0, 0)),
            pl.BlockSpec((1, H), lambda i: (0, 0)),
        ],
        out_specs=pl.BlockSpec((bn, H), lambda i: (i, 0)),
        out_shape=jax.ShapeDtypeStruct((N, H), jnp.float32),
    )(p, v, w1t, b1r, w2t, b2r)


def kernel(v, e, edge_index, W1, b1, W2, b2):
    idx = edge_index[1].reshape(NW, NCH, C)
    zrows = jnp.zeros((RPT, F), jnp.float32)
    partials = _sc_segment_sum(e, idx, zrows)
    return _mlp(
        partials,
        v,
        W1.T,
        b1.reshape(1, H),
        W2.T,
        b2.reshape(1, H),
    )
